# baseline (speedup 1.0000x reference)
import functools

import jax
import jax.numpy as jnp
import numpy as np
from jax import lax
from jax.experimental import pallas as pl
from jax.experimental.pallas import tpu as pltpu
from jax.experimental.pallas import tpu_sc as plsc

N = 10000
NPAD = 10240
E = 160000
EPAD = 163840
DUMP = N
D = 256
H = 128
NC, NS = 2, 16
BN = 1000
S = 4548
SP = 4608

_rng = np.random.RandomState(0)
_ROW = _rng.randint(0, S, size=S).astype(np.int32)
_COL = _rng.randint(0, S, size=S).astype(np.int32)


def _pad_idx(a):
    return np.concatenate([a.astype(np.int32), np.zeros(SP - S, np.int32)])


_IDXS = np.concatenate([_pad_idx(_ROW), _pad_idx(_COL)]).reshape(32, 3, 96)
_IDXP = np.stack([
    np.concatenate([_pad_idx(_ROW).reshape(16, 3, 96)[s],
                    _pad_idx(_COL).reshape(16, 3, 96)[s]])
    for s in range(16)
])

@functools.cache
def _mesh():
    return plsc.VectorSubcoreMesh(core_axis_name="c", subcore_axis_name="s")


def _deg_body(dst_hbm, ones_hbm, zero_hbm, out_hbm, dst_v, ones_v, acc, sem):
    c = lax.axis_index("c")
    s = lax.axis_index("s")
    w = c * NS + s
    pltpu.sync_copy(zero_hbm, acc.at[pl.ds(s * 640, 640)])
    pltpu.sync_copy(ones_hbm, ones_v)
    pltpu.sync_copy(dst_hbm.at[w], dst_v)
    plsc.subcore_barrier()

    def fire(j, carry):
        pltpu.async_copy(ones_v, acc.at[dst_v.at[j]], sem, add=True)
        return carry

    lax.fori_loop(0, 40, fire, 0)

    def drain(j, carry):
        pltpu.make_async_copy(ones_v, acc.at[dst_v.at[j]], sem).wait()
        return carry

    lax.fori_loop(0, 40, drain, 0)
    plsc.subcore_barrier()
    pltpu.sync_copy(acc.at[pl.ds(s * 640, 640)],
                    out_hbm.at[c, pl.ds(s * 640, 640)])


@functools.cache
def _deg_call():
    return pl.kernel(
        _deg_body,
        out_type=jax.ShapeDtypeStruct((NC, NPAD, H), jnp.float32),
        mesh=_mesh(),
        scratch_types=[
            pltpu.VMEM((40, 128), jnp.int32),
            pltpu.VMEM((128, H), jnp.float32),
            pltpu.VMEM_SHARED((NPAD, H), jnp.float32),
            pltpu.SemaphoreType.DMA,
        ],
    )


def _scat_body(gtab_hbm, src_hbm, dst_hbm, out_hbm,
               src_v, dst_v, buf0, buf1, sem0, sem1, acc):
    c = lax.axis_index("c")
    s = lax.axis_index("s")

    def gidx(j):
        return plsc.Indices(src_v.at[j], ignored_value=-1)

    def start_gather(j, buf, sem):
        pltpu.async_copy(gtab_hbm.at[gidx(j)], buf, sem)

    def wait_gather(j, buf, sem):
        pltpu.make_async_copy(gtab_hbm.at[gidx(j)], buf, sem).wait()

    def scat(j, buf):
        pltpu.sync_copy(
            buf, acc.at[plsc.Indices(dst_v.at[j], ignored_value=-1)], add=True)

    pltpu.sync_copy(gtab_hbm.at[pl.ds(c * NPAD + s * 640, 640)],
                    acc.at[pl.ds(s * 640, 640)])

    def body(jj, carry):
        j = 2 * jj
        wait_gather(j, buf0, sem0)
        start_gather(j + 1, buf1, sem1)
        scat(j, buf0)
        wait_gather(j + 1, buf1, sem1)

        @pl.when(jj < 19)
        def _():
            start_gather(j + 2, buf0, sem0)

        scat(j + 1, buf1)
        return carry

    for hh in range(2):
        pltpu.sync_copy(src_hbm.at[c, s, pl.ds(hh * 40, 40)], src_v)
        pltpu.sync_copy(dst_hbm.at[s, pl.ds(hh * 40, 40)], dst_v)
        start_gather(0, buf0, sem0)
        if hh == 0:
            plsc.subcore_barrier()
        lax.fori_loop(0, 20, body, 0)
    plsc.subcore_barrier()
    pltpu.sync_copy(acc.at[pl.ds(s * 640, 640)],
                    out_hbm.at[pl.ds(c * NPAD + s * 640, 640)])


@functools.cache
def _scat_call():
    return pl.kernel(
        _scat_body,
        out_type=jax.ShapeDtypeStruct((2 * NPAD, H), jnp.float32),
        mesh=_mesh(),
        scratch_types=[
            pltpu.VMEM((40, 128), jnp.int32),
            pltpu.VMEM((40, 128), jnp.int32),
            pltpu.VMEM((128, H), jnp.float32),
            pltpu.VMEM((128, H), jnp.float32),
            pltpu.SemaphoreType.DMA,
            pltpu.SemaphoreType.DMA,
            pltpu.VMEM_SHARED((NPAD, H), jnp.float32),
        ],
    )


def _scat2_body(gtab_hbm, src_hbm, dst_hbm, idxp_hbm, idxs_hbm, ptab_hbm,
                rows_hbm, pout_hbm,
                src_v, dst_v, buf0, buf1, idxp_v, idxs_v, sem0, sem1, acc):
    c = lax.axis_index("c")
    s = lax.axis_index("s")
    w = c * NS + s

    def gidx(j):
        return plsc.Indices(src_v.at[j], ignored_value=-1)

    def start_gather(j, buf, sem):
        pltpu.async_copy(gtab_hbm.at[gidx(j)], buf, sem)

    def wait_gather(j, buf, sem):
        pltpu.make_async_copy(gtab_hbm.at[gidx(j)], buf, sem).wait()

    def scat(j, buf):
        pltpu.sync_copy(
            buf, acc.at[plsc.Indices(dst_v.at[j], ignored_value=-1)], add=True)

    @pl.when(s < 8)
    def _():
        pltpu.sync_copy(gtab_hbm.at[pl.ds(c * NPAD + s * 640, 640)],
                        acc.at[pl.ds(s * 640, 640)])

    def body(jj, carry):
        j = 2 * jj
        wait_gather(j, buf0, sem0)
        start_gather(j + 1, buf1, sem1)
        scat(j, buf0)
        wait_gather(j + 1, buf1, sem1)

        @pl.when(jj < 19)
        def _():
            start_gather(j + 2, buf0, sem0)

        scat(j + 1, buf1)
        return carry

    for hh in range(2):
        pltpu.sync_copy(src_hbm.at[c, s, pl.ds(hh * 40, 40)], src_v)
        pltpu.sync_copy(dst_hbm.at[s, pl.ds(hh * 40, 40)], dst_v)
        start_gather(0, buf0, sem0)
        if hh == 0:
            plsc.subcore_barrier()
        lax.fori_loop(0, 20, body, 0)
    plsc.subcore_barrier()

    pltpu.sync_copy(idxp_hbm.at[s], idxp_v)
    pltpu.sync_copy(idxs_hbm.at[w], idxs_v)

    def pairs(j, carry):
        pltpu.async_copy(acc.at[idxp_v.at[j]], buf0.at[pl.ds(0, 96)],
                         sem0).wait()
        off = c * (2 * SP) + (j // 3) * SP + s * 288 + (j % 3) * 96
        pltpu.sync_copy(buf0.at[pl.ds(0, 96)], rows_hbm.at[pl.ds(off, 96)])
        return carry

    lax.fori_loop(0, 6, pairs, 0)

    def scals(j, carry):
        pltpu.async_copy(ptab_hbm.at[idxs_v.at[j]], buf1.at[pl.ds(0, 96)],
                         sem1).wait()
        pltpu.sync_copy(buf1.at[pl.ds(0, 96)],
                        pout_hbm.at[pl.ds(w * 288 + j * 96, 96)])
        return carry

    lax.fori_loop(0, 3, scals, 0)


@functools.cache
def _scat2_call():
    return pl.kernel(
        _scat2_body,
        out_type=[
            jax.ShapeDtypeStruct((4 * SP, H), jnp.float32),
            jax.ShapeDtypeStruct((2 * SP, H), jnp.float32),
        ],
        mesh=_mesh(),
        scratch_types=[
            pltpu.VMEM((40, 128), jnp.int32),
            pltpu.VMEM((40, 128), jnp.int32),
            pltpu.VMEM((128, H), jnp.float32),
            pltpu.VMEM((128, H), jnp.float32),
            pltpu.VMEM((6, 96), jnp.int32),
            pltpu.VMEM((3, 96), jnp.int32),
            pltpu.SemaphoreType.DMA,
            pltpu.SemaphoreType.DMA,
            pltpu.VMEM_SHARED((NPAD, H), jnp.float32),
        ],
    )


def _mm1a_body(x_ref, w_ref, b_ref, h_ref):
    h = jnp.dot(x_ref[...], w_ref[...], preferred_element_type=jnp.float32)
    h_ref[...] = (h + b_ref[...])[None]


def _mm1a(x, W1, b1):
    return pl.pallas_call(
        _mm1a_body,
        grid=(N // BN, 2),
        in_specs=[
            pl.BlockSpec((BN, D), lambda i, j: (i, 0)),
            pl.BlockSpec((D, H), lambda i, j: (0, j)),
            pl.BlockSpec((1, H), lambda i, j: (0, j)),
        ],
        out_specs=pl.BlockSpec((1, BN, H), lambda i, j: (j, i, 0)),
        out_shape=jax.ShapeDtypeStruct((NC, NPAD, H), jnp.float32),
    )(x, W1, b1.reshape(1, D))


def _mm1b_body(h_ref, p0_ref, p1_ref, g_ref, dinv_ref):
    deg = 1.0 + p0_ref[...] + p1_ref[...]
    dinv = lax.rsqrt(deg)
    g_ref[...] = (dinv * h_ref[0])[None]
    dinv_ref[...] = dinv


def _mm1b(h_packed, p0, p1):
    return pl.pallas_call(
        _mm1b_body,
        grid=(N // BN, 2),
        in_specs=[
            pl.BlockSpec((1, BN, H), lambda i, j: (j, i, 0)),
            pl.BlockSpec((BN, 1), lambda i, j: (i, 0)),
            pl.BlockSpec((BN, 1), lambda i, j: (i, 0)),
        ],
        out_specs=[
            pl.BlockSpec((1, BN, H), lambda i, j: (j, i, 0)),
            pl.BlockSpec((BN, 1), lambda i, j: (i, 0)),
        ],
        out_shape=[
            jax.ShapeDtypeStruct((NC, NPAD, H), jnp.float32),
            jax.ShapeDtypeStruct((N, 1), jnp.float32),
        ],
    )(h_packed, p0, p1)


def _mm2_body(alo_ref, ahi_ref, dinv_ref, w_ref, b_ref, g_ref):
    dinv = dinv_ref[...]
    a = jnp.concatenate([alo_ref[0], ahi_ref[0]], axis=1)
    a = jnp.maximum(dinv * a, 0.0)
    h = jnp.dot(a, w_ref[...], preferred_element_type=jnp.float32)
    g_ref[...] = (dinv * (h + b_ref[...]))[None]


def _mm2(acc1, dinv, W2, b2):
    return pl.pallas_call(
        _mm2_body,
        grid=(N // BN, 2),
        in_specs=[
            pl.BlockSpec((1, BN, H), lambda i, j: (0, i, 0)),
            pl.BlockSpec((1, BN, H), lambda i, j: (1, i, 0)),
            pl.BlockSpec((BN, 1), lambda i, j: (i, 0)),
            pl.BlockSpec((D, H), lambda i, j: (0, j)),
            pl.BlockSpec((1, H), lambda i, j: (0, j)),
        ],
        out_specs=pl.BlockSpec((1, BN, H), lambda i, j: (j, i, 0)),
        out_shape=jax.ShapeDtypeStruct((NC, NPAD, H), jnp.float32),
    )(acc1, acc1, dinv, W2, b2.reshape(1, D))


def _heads_body(ilo, ihi, jlo, jhi, dvr, dvc, yr, yc,
                wd1, bd1, wd2, bd2, we1, be1, we2, be2, o_dire, o_dist):
    valid = (lax.broadcasted_iota(jnp.int32, (SP, 1), 0) < S).astype(jnp.float32)
    emb_i = jnp.maximum(dvr[...] * jnp.concatenate([ilo[...], ihi[...]], axis=1), 0.0)
    emb_j = jnp.maximum(dvc[...] * jnp.concatenate([jlo[...], jhi[...]], axis=1), 0.0)
    e = emb_i - emb_j + emb_i * emb_j
    diff = yr[...] - yc[...]

    def head(feat, w1, b1_, w2, b2_, ncls, labels):
        h1 = jnp.maximum(jnp.dot(feat, w1[...], preferred_element_type=jnp.float32)
                         + b1_[...], 0.0)
        h2 = jnp.dot(h1, w2[...], preferred_element_type=jnp.float32) + b2_[...]
        m = jnp.max(h2, axis=1, keepdims=True)
        ex = jnp.exp(h2 - m)
        p = ex / jnp.sum(ex, axis=1, keepdims=True)
        m2 = jnp.max(p, axis=1, keepdims=True)
        lse = m2 + jnp.log(jnp.sum(jnp.exp(p - m2), axis=1, keepdims=True))
        oh = (lax.broadcasted_iota(jnp.int32, (SP, ncls), 1) == labels).astype(jnp.float32)
        take = jnp.sum(p * oh, axis=1, keepdims=True)
        return (-jnp.sum((take - lse) * valid) * (1.0 / S)).reshape(1, 1)

    lab_dire = jnp.where(diff < 0, 0, jnp.where(diff == 0, 1, 2))
    o_dire[...] = head(e, we1, be1, we2, be2, 3, lab_dire)
    lab_dist = jnp.abs(diff)
    o_dist[...] = head(jnp.abs(e), wd1, bd1, wd2, bd2, 4, lab_dist)


def _heads(ilo, ihi, jlo, jhi, dvr, dvc, yr, yc, Wd1, bd1, Wd2, bd2, We1, be1, We2, be2):
    return pl.pallas_call(
        _heads_body,
        out_shape=[
            jax.ShapeDtypeStruct((1, 1), jnp.float32),
            jax.ShapeDtypeStruct((1, 1), jnp.float32),
        ],
    )(ilo, ihi, jlo, jhi, dvr, dvc, yr, yc,
      Wd1, bd1.reshape(1, D), Wd2, bd2.reshape(1, 4),
      We1, be1.reshape(1, D), We2, be2.reshape(1, 3))


def kernel(x, y, edge_index, W1, b1, W2, b2, Wd1, bd1, Wd2, bd2, We1, be1, We2, be2):
    src = edge_index[0]
    dst = edge_index[1]

    pad_src = jnp.zeros((EPAD - E,), src.dtype)
    pad_dst = jnp.full((EPAD - E,), DUMP, dst.dtype)
    src_p = jnp.concatenate([src, pad_src])
    dst_p = jnp.concatenate([dst, pad_dst])
    valid1 = dst_p < N
    src2_adj = jnp.stack([src_p, src_p + NPAD])
    valid2 = (dst_p < S)[None, :]
    src_adj = jnp.where(valid1[None, :], src2_adj, -1).reshape(NC, NS, 80, 128)
    dst_sc = jnp.where(valid1, dst_p, -1).reshape(NS, 80, 128)
    src_adj2 = jnp.where(valid2, src2_adj, -1).reshape(NC, NS, 80, 128)
    dst_sc2 = jnp.where(valid2[0], dst_p, -1).reshape(NS, 80, 128)
    dst_deg = dst_p.reshape(32, 40, 128)
    ones_rows = jnp.zeros((128, H), jnp.float32).at[:, 0].set(1.0)
    zero_rows = jnp.zeros((640, H), jnp.float32)
    idxs = jnp.asarray(_IDXS)

    deg_parts = _deg_call()(dst_deg, ones_rows, zero_rows)
    p0 = deg_parts[0, :N, 0].reshape(N, 1)
    p1 = deg_parts[1, :N, 0].reshape(N, 1)

    h1p = _mm1a(x, W1, b1)
    g1, dinv = _mm1b(h1p, p0, p1)

    acc1 = _scat_call()(g1.reshape(2 * NPAD, H), src_adj, dst_sc)

    g2 = _mm2(acc1.reshape(NC, NPAD, H), dinv, W2, b2)

    yf = jax.lax.bitcast_convert_type(y, jnp.float32).reshape(N, 1)
    ptab = jnp.concatenate(
        [dinv, yf, jnp.zeros((N, 126), jnp.float32)], axis=1)
    rows, pout = _scat2_call()(g2.reshape(2 * NPAD, H), src_adj2, dst_sc2,
                               jnp.asarray(_IDXP), idxs, ptab)

    ilo = rows[0 * SP:1 * SP]
    jlo = rows[1 * SP:2 * SP]
    ihi = rows[2 * SP:3 * SP]
    jhi = rows[3 * SP:4 * SP]
    dvr = pout[:SP, 0:1]
    dvc = pout[SP:, 0:1]
    yr = jax.lax.bitcast_convert_type(pout[:SP, 1:2], jnp.int32)
    yc = jax.lax.bitcast_convert_type(pout[SP:, 1:2], jnp.int32)
    loss_dire, loss_dist = _heads(ilo, ihi, jlo, jhi, dvr, dvc, yr, yc,
                                  Wd1, bd1, Wd2, bd2, We1, be1, We2, be2)
    return (loss_dire[0, 0], loss_dist[0, 0])
